# baseline (speedup 1.0000x reference)
import functools

import jax
import jax.numpy as jnp
from jax import lax
from jax.experimental import pallas as pl
from jax.experimental.pallas import tpu as pltpu
from jax.experimental.pallas import tpu_sc as plsc

BATCH = 16384
EMB = 8
NC = 2
NS = 16
NW = NC * NS
B_PER_W = BATCH // NW
CHUNK = 128
N_CHUNKS = B_PER_W // CHUNK


def _gather_body(uidx_hbm, iidx_hbm, utab_hbm, itab_hbm, uout_hbm, iout_hbm,
                 uidx_v, iidx_v, urows_v, irows_v, sem):
    wid = lax.axis_index("s") * NC + lax.axis_index("c")
    base = wid * B_PER_W
    pltpu.sync_copy(uidx_hbm.at[pl.ds(base, B_PER_W)], uidx_v)
    pltpu.sync_copy(iidx_hbm.at[pl.ds(base, B_PER_W)], iidx_v)
    for c in range(N_CHUNKS):
        sl = pl.ds(c * CHUNK, CHUNK)
        pltpu.async_copy(utab_hbm.at[uidx_v.at[sl]], urows_v.at[sl], sem).wait()
        pltpu.async_copy(itab_hbm.at[iidx_v.at[sl]], irows_v.at[sl], sem).wait()
    pltpu.sync_copy(urows_v, uout_hbm.at[pl.ds(base, B_PER_W)])
    pltpu.sync_copy(irows_v, iout_hbm.at[pl.ds(base, B_PER_W)])


@functools.lru_cache(maxsize=None)
def _sc_gather():
    return pl.kernel(
        _gather_body,
        out_type=(
            jax.ShapeDtypeStruct((BATCH, EMB), jnp.float32),
            jax.ShapeDtypeStruct((BATCH, EMB), jnp.float32),
        ),
        mesh=plsc.VectorSubcoreMesh(core_axis_name="c", subcore_axis_name="s"),
        scratch_types=[
            pltpu.VMEM((B_PER_W,), jnp.int32),
            pltpu.VMEM((B_PER_W,), jnp.int32),
            pltpu.VMEM((B_PER_W, EMB), jnp.float32),
            pltpu.VMEM((B_PER_W, EMB), jnp.float32),
            pltpu.SemaphoreType.DMA,
        ],
        compiler_params=pltpu.CompilerParams(use_tc_tiling_on_sc=False),
    )

BLK = 2048


def _mlp_body(u_ref, v_ref, w1u_ref, w1v_ref, b1_ref, w2_ref, b2_ref,
              w3_ref, b3_ref, wo_ref, bo_ref, out_ref):
    u = u_ref[...]
    v = v_ref[...]
    h = u @ w1u_ref[...] + v @ w1v_ref[...] + b1_ref[...]
    h = jnp.maximum(h, 0.0)
    h = jnp.maximum(h @ w2_ref[...] + b2_ref[...], 0.0)
    h = jnp.maximum(h @ w3_ref[...] + b3_ref[...], 0.0)
    z = h @ wo_ref[...] + bo_ref[...]
    out_ref[...] = jax.nn.sigmoid(z)


@jax.jit
def kernel(user_input, item_input, user_table, item_table,
           W1, b1, W2, b2, W3, b3, Wo, bo):
    u_emb, i_emb = _sc_gather()(user_input, item_input, user_table, item_table)

    w1u = W1[:EMB]
    w1v = W1[EMB:]
    grid = (BATCH // BLK,)
    rep = lambda i: (0, 0)
    pred = pl.pallas_call(
        _mlp_body,
        grid=grid,
        in_specs=[
            pl.BlockSpec((BLK, EMB), lambda i: (i, 0)),
            pl.BlockSpec((BLK, EMB), lambda i: (i, 0)),
            pl.BlockSpec((EMB, 64), rep),
            pl.BlockSpec((EMB, 64), rep),
            pl.BlockSpec((1, 64), rep),
            pl.BlockSpec((64, 128), rep),
            pl.BlockSpec((1, 128), rep),
            pl.BlockSpec((128, 32), rep),
            pl.BlockSpec((1, 32), rep),
            pl.BlockSpec((32, 1), rep),
            pl.BlockSpec((1, 1), rep),
        ],
        out_specs=pl.BlockSpec((BLK, 1), lambda i: (i, 0)),
        out_shape=jax.ShapeDtypeStruct((BATCH, 1), jnp.float32),
    )(
        u_emb, i_emb, w1u, w1v, b1.reshape(1, 64), W2, b2.reshape(1, 128),
        W3, b3.reshape(1, 32), Wo, bo.reshape(1, 1),
    )
    return pred
